# baseline (speedup 1.0000x reference)
import functools

import jax
import jax.numpy as jnp
from jax import lax
from jax.experimental import pallas as pl
from jax.experimental.pallas import tpu as pltpu
from jax.experimental.pallas import tpu_sc as plsc

VOCAB = 1000000
DIM = 64
SEQ = 200
BATCH = 4096

NUM_CORES = 2
NUM_SUBCORES = 16
NW = NUM_CORES * NUM_SUBCORES
BLK = BATCH // NW
SHI = SEQ // 8

_mesh = plsc.VectorSubcoreMesh(
    core_axis_name="c", subcore_axis_name="s",
    num_cores=NUM_CORES, num_subcores=NUM_SUBCORES)


@functools.partial(
    pl.kernel,
    out_type=[
        jax.ShapeDtypeStruct((SEQ, DIM // 8, NW, 8, BLK), jnp.float32),
        jax.ShapeDtypeStruct((BATCH,), jnp.int32),
    ],
    mesh=_mesh,
    compiler_params=pltpu.CompilerParams(
        needs_layout_passes=False, use_tc_tiling_on_sc=False),
    scratch_types=[
        pltpu.VMEM((SEQ, BLK), jnp.int32),
        pltpu.VMEM((SEQ, DIM), jnp.float32),
        pltpu.VMEM((BLK, DIM), jnp.float32),
        pltpu.VMEM((BLK, DIM), jnp.float32),
        pltpu.VMEM((DIM, BLK), jnp.float32),
        pltpu.VMEM((DIM, BLK), jnp.float32),
        pltpu.VMEM((BLK,), jnp.int32),
        pltpu.SemaphoreType.DMA,
        pltpu.SemaphoreType.DMA,
        pltpu.SemaphoreType.DMA,
        pltpu.SemaphoreType.DMA,
        pltpu.SemaphoreType.DMA,
    ],
)
def _emb_kernel(table_hbm, tok_hbm, pe_hbm, out_hbm, pad_hbm,
                idx_v, pe_v, in0, in1, out0, out1, pad_v,
                stsem, gsem0, gsem1, osem0, osem1):
    cid = lax.axis_index("c")
    sid = lax.axis_index("s")
    wid = sid * NUM_CORES + cid

    for shi in range(SHI):
        pltpu.make_async_copy(
            tok_hbm.at[shi, wid], idx_v.at[pl.ds(shi * 8, 8)], stsem).start()
    for shi in range(SHI):
        pltpu.make_async_copy(
            tok_hbm.at[shi, wid], idx_v.at[pl.ds(shi * 8, 8)], stsem).wait()

    def gather_start(s, buf, sem):
        pltpu.make_async_copy(table_hbm.at[idx_v.at[s]], buf, sem).start()

    def gather_wait(s, buf, sem):
        pltpu.make_async_copy(table_hbm.at[idx_v.at[s]], buf, sem).wait()

    def out_start(s, buf, sem):
        for dhi in range(DIM // 8):
            pltpu.make_async_copy(
                buf.at[pl.ds(dhi * 8, 8)], out_hbm.at[s, dhi, wid], sem).start()

    def out_wait(s, buf, sem):
        for dhi in range(DIM // 8):
            pltpu.make_async_copy(
                buf.at[pl.ds(dhi * 8, 8)], out_hbm.at[s, dhi, wid], sem).wait()

    gather_start(0, in0, gsem0)
    gather_start(1, in1, gsem1)

    pltpu.sync_copy(pe_hbm, pe_v)

    def pad_body(s, best):
        new = []
        for g in range(BLK // 16):
            tok = idx_v[s, pl.ds(16 * g, 16)]
            new.append(jnp.maximum(best[g], jnp.where(tok != 0, s, 0)))
        return tuple(new)

    best = lax.fori_loop(
        0, SEQ, pad_body,
        tuple(jnp.zeros((16,), jnp.int32) for _ in range(BLK // 16)))
    for g in range(BLK // 16):
        pad_v[pl.ds(16 * g, 16)] = best[g] + 1
    pltpu.sync_copy(pad_v, pad_hbm.at[pl.ds(wid * BLK, BLK)])

    iota16 = lax.iota(jnp.int32, 16)

    def compute(s, inb, outb):
        s_splat = jnp.full((16,), s, jnp.int32)

        def d_body(d, _):
            col = jnp.full((16,), d, jnp.int32)
            pe_b = plsc.load_gather(pe_v, [s_splat, col])
            for g in range(BLK // 16):
                row = 16 * g + iota16
                val = plsc.load_gather(inb, [row, col])
                outb[d, pl.ds(16 * g, 16)] = val * 8.0 + pe_b
            return 0
        lax.fori_loop(0, DIM, d_body, 0)

    def s_phase(s, inb, outb, gsem, osem):
        gather_wait(s, inb, gsem)

        @pl.when(s >= 2)
        def _():
            out_wait(s, outb, osem)

        compute(s, inb, outb)

        @pl.when(s + 2 < SEQ)
        def _():
            gather_start(s + 2, inb, gsem)

        out_start(s, outb, osem)

    def loop_body(ss, _):
        s_phase(2 * ss, in0, out0, gsem0, osem0)
        s_phase(2 * ss + 1, in1, out1, gsem1, osem1)
        return 0

    lax.fori_loop(0, SEQ // 2, loop_body, 0)

    out_wait(SEQ - 2, out0, osem0)
    out_wait(SEQ - 1, out1, osem1)


def kernel(token_tensor, table, PE):
    tok4d = (token_tensor.astype(jnp.int32).T
             .reshape(SHI, 8, NW, BLK).transpose(0, 2, 1, 3))
    out5d, pad_idxs = _emb_kernel(table, tok4d, PE)
    out = out5d.transpose(2, 4, 0, 1, 3).reshape(BATCH, SEQ, DIM)
    return out, pad_idxs


# speedup vs baseline: 1.4592x; 1.4592x over previous
import functools

import jax
import jax.numpy as jnp
from jax import lax
from jax.experimental import pallas as pl
from jax.experimental.pallas import tpu as pltpu
from jax.experimental.pallas import tpu_sc as plsc

VOCAB = 1000000
DIM = 64
SEQ = 200
BATCH = 4096

NUM_CORES = 2
NUM_SUBCORES = 16
NW = NUM_CORES * NUM_SUBCORES
BLK = BATCH // NW
SHI = SEQ // 8

_mesh = plsc.VectorSubcoreMesh(
    core_axis_name="c", subcore_axis_name="s",
    num_cores=NUM_CORES, num_subcores=NUM_SUBCORES)


@functools.partial(
    pl.kernel,
    out_type=[
        jax.ShapeDtypeStruct((SEQ, DIM // 8, NW, 8, BLK), jnp.float32),
        jax.ShapeDtypeStruct((BATCH,), jnp.int32),
    ],
    mesh=_mesh,
    compiler_params=pltpu.CompilerParams(
        needs_layout_passes=False, use_tc_tiling_on_sc=False),
    scratch_types=[
        pltpu.VMEM((SEQ, BLK), jnp.int32),
        pltpu.VMEM((SEQ, DIM), jnp.float32),
        pltpu.VMEM((BLK, DIM), jnp.float32),
        pltpu.VMEM((BLK, DIM), jnp.float32),
        pltpu.VMEM((DIM, BLK), jnp.float32),
        pltpu.VMEM((DIM, BLK), jnp.float32),
        pltpu.VMEM((BLK,), jnp.int32),
        pltpu.SemaphoreType.DMA,
        pltpu.SemaphoreType.DMA,
        pltpu.SemaphoreType.DMA,
        pltpu.SemaphoreType.DMA,
        pltpu.SemaphoreType.DMA,
    ],
)
def _emb_kernel(table_hbm, tok_hbm, pe_hbm, out_hbm, pad_hbm,
                idx_v, pe_v, in0, in1, out0, out1, pad_v,
                stsem, gsem0, gsem1, osem0, osem1):
    cid = lax.axis_index("c")
    sid = lax.axis_index("s")
    wid = sid * NUM_CORES + cid

    for shi in range(SHI):
        pltpu.make_async_copy(
            tok_hbm.at[shi, wid], idx_v.at[pl.ds(shi * 8, 8)], stsem).start()
    for shi in range(SHI):
        pltpu.make_async_copy(
            tok_hbm.at[shi, wid], idx_v.at[pl.ds(shi * 8, 8)], stsem).wait()

    def gather_start(s, buf, sem):
        pltpu.make_async_copy(table_hbm.at[idx_v.at[s]], buf, sem).start()

    def gather_wait(s, buf, sem):
        pltpu.make_async_copy(table_hbm.at[idx_v.at[s]], buf, sem).wait()

    def out_start(s, buf, sem):
        for dhi in range(DIM // 8):
            pltpu.make_async_copy(
                buf.at[pl.ds(dhi * 8, 8)], out_hbm.at[s, dhi, wid], sem).start()

    def out_wait(s, buf, sem):
        for dhi in range(DIM // 8):
            pltpu.make_async_copy(
                buf.at[pl.ds(dhi * 8, 8)], out_hbm.at[s, dhi, wid], sem).wait()

    gather_start(0, in0, gsem0)
    gather_start(1, in1, gsem1)

    pltpu.sync_copy(pe_hbm, pe_v)

    def pad_body(s, best):
        new = []
        for g in range(BLK // 16):
            tok = idx_v[s, pl.ds(16 * g, 16)]
            new.append(jnp.maximum(best[g], jnp.where(tok != 0, s, 0)))
        return tuple(new)

    best = lax.fori_loop(
        0, SEQ, pad_body,
        tuple(jnp.zeros((16,), jnp.int32) for _ in range(BLK // 16)))
    for g in range(BLK // 16):
        pad_v[pl.ds(16 * g, 16)] = best[g] + 1
    pltpu.sync_copy(pad_v, pad_hbm.at[pl.ds(wid * BLK, BLK)])

    iota16 = lax.iota(jnp.int32, 16)

    def compute(s, inb, outb):
        s_splat = jnp.full((16,), s, jnp.int32)

        @plsc.parallel_loop(0, DIM, 1, unroll=8)
        def d_body(d):
            col = jnp.full((16,), d, jnp.int32)
            pe_b = plsc.load_gather(pe_v, [s_splat, col])
            for g in range(BLK // 16):
                row = 16 * g + iota16
                val = plsc.load_gather(inb, [row, col])
                outb[d, pl.ds(16 * g, 16)] = val * 8.0 + pe_b

    def s_phase(s, inb, outb, gsem, osem):
        gather_wait(s, inb, gsem)

        @pl.when(s >= 2)
        def _():
            out_wait(s, outb, osem)

        compute(s, inb, outb)

        @pl.when(s + 2 < SEQ)
        def _():
            gather_start(s + 2, inb, gsem)

        out_start(s, outb, osem)

    def loop_body(ss, _):
        s_phase(2 * ss, in0, out0, gsem0, osem0)
        s_phase(2 * ss + 1, in1, out1, gsem1, osem1)
        return 0

    lax.fori_loop(0, SEQ // 2, loop_body, 0)

    out_wait(SEQ - 2, out0, osem0)
    out_wait(SEQ - 1, out1, osem1)


def kernel(token_tensor, table, PE):
    tok4d = (token_tensor.astype(jnp.int32).T
             .reshape(SHI, 8, NW, BLK).transpose(0, 2, 1, 3))
    out5d, pad_idxs = _emb_kernel(table, tok4d, PE)
    out = out5d.transpose(2, 4, 0, 1, 3).reshape(BATCH, SEQ, DIM)
    return out, pad_idxs


# speedup vs baseline: 2.4139x; 1.6542x over previous
import functools

import jax
import jax.numpy as jnp
from jax import lax
from jax.experimental import pallas as pl
from jax.experimental.pallas import tpu as pltpu
from jax.experimental.pallas import tpu_sc as plsc

VOCAB = 1000000
DIM = 64
SEQ = 200
BATCH = 4096

NUM_CORES = 2
NUM_SUBCORES = 16
NW = NUM_CORES * NUM_SUBCORES
BLK = BATCH // NW
SHI = SEQ // 8

_mesh = plsc.VectorSubcoreMesh(
    core_axis_name="c", subcore_axis_name="s",
    num_cores=NUM_CORES, num_subcores=NUM_SUBCORES)


@functools.partial(
    pl.kernel,
    out_type=[
        jax.ShapeDtypeStruct((SEQ, DIM // 8, NW, 8, BLK), jnp.float32),
        jax.ShapeDtypeStruct((BATCH,), jnp.int32),
    ],
    mesh=_mesh,
    compiler_params=pltpu.CompilerParams(
        needs_layout_passes=False, use_tc_tiling_on_sc=False),
    scratch_types=[
        pltpu.VMEM((SEQ, BLK), jnp.int32),
        pltpu.VMEM((SEQ, DIM), jnp.float32),
        pltpu.VMEM((BLK, DIM), jnp.float32),
        pltpu.VMEM((BLK, DIM), jnp.float32),
        pltpu.VMEM((BLK, DIM + 1), jnp.float32),
        pltpu.VMEM((DIM, BLK), jnp.float32),
        pltpu.VMEM((DIM, BLK), jnp.float32),
        pltpu.VMEM((BLK,), jnp.int32),
        pltpu.SemaphoreType.DMA,
        pltpu.SemaphoreType.DMA,
        pltpu.SemaphoreType.DMA,
        pltpu.SemaphoreType.DMA,
        pltpu.SemaphoreType.DMA,
    ],
)
def _emb_kernel(table_hbm, tok_hbm, pe_hbm, out_hbm, pad_hbm,
                idx_v, pe_v, in0, in1, inp, out0, out1, pad_v,
                stsem, gsem0, gsem1, osem0, osem1):
    cid = lax.axis_index("c")
    sid = lax.axis_index("s")
    wid = sid * NUM_CORES + cid

    for shi in range(SHI):
        pltpu.make_async_copy(
            tok_hbm.at[shi, wid], idx_v.at[pl.ds(shi * 8, 8)], stsem).start()
    for shi in range(SHI):
        pltpu.make_async_copy(
            tok_hbm.at[shi, wid], idx_v.at[pl.ds(shi * 8, 8)], stsem).wait()

    def gather_start(s, buf, sem):
        pltpu.make_async_copy(table_hbm.at[idx_v.at[s]], buf, sem).start()

    def gather_wait(s, buf, sem):
        pltpu.make_async_copy(table_hbm.at[idx_v.at[s]], buf, sem).wait()

    def out_start(s, buf, sem):
        for dhi in range(DIM // 8):
            pltpu.make_async_copy(
                buf.at[pl.ds(dhi * 8, 8)], out_hbm.at[s, dhi, wid], sem).start()

    def out_wait(s, buf, sem):
        for dhi in range(DIM // 8):
            pltpu.make_async_copy(
                buf.at[pl.ds(dhi * 8, 8)], out_hbm.at[s, dhi, wid], sem).wait()

    gather_start(0, in0, gsem0)
    gather_start(1, in1, gsem1)

    pltpu.sync_copy(pe_hbm, pe_v)

    def pad_body(s, best):
        new = []
        for g in range(BLK // 16):
            tok = idx_v[s, pl.ds(16 * g, 16)]
            new.append(jnp.maximum(best[g], jnp.where(tok != 0, s, 0)))
        return tuple(new)

    best = lax.fori_loop(
        0, SEQ, pad_body,
        tuple(jnp.zeros((16,), jnp.int32) for _ in range(BLK // 16)))
    for g in range(BLK // 16):
        pad_v[pl.ds(16 * g, 16)] = best[g] + 1
    pltpu.sync_copy(pad_v, pad_hbm.at[pl.ds(wid * BLK, BLK)])

    iota16 = lax.iota(jnp.int32, 16)

    def compute(s, inb, outb):
        pe_c = [pe_v[s, pl.ds(16 * c, 16)] for c in range(DIM // 16)]

        @plsc.parallel_loop(0, BLK, 1, unroll=8)
        def r_body(r):
            for c in range(DIM // 16):
                sl = pl.ds(16 * c, 16)
                inp[r, sl] = inb[r, sl] * 8.0 + pe_c[c]

        @plsc.parallel_loop(0, DIM, 1, unroll=8)
        def d_body(d):
            col = jnp.full((16,), d, jnp.int32)
            for g in range(BLK // 16):
                row = 16 * g + iota16
                outb[d, pl.ds(16 * g, 16)] = plsc.load_gather(inp, [row, col])

    def s_phase(s, inb, outb, gsem, osem):
        gather_wait(s, inb, gsem)

        @pl.when(s >= 2)
        def _():
            out_wait(s, outb, osem)

        compute(s, inb, outb)

        @pl.when(s + 2 < SEQ)
        def _():
            gather_start(s + 2, inb, gsem)

        out_start(s, outb, osem)

    def loop_body(ss, _):
        s_phase(2 * ss, in0, out0, gsem0, osem0)
        s_phase(2 * ss + 1, in1, out1, gsem1, osem1)
        return 0

    lax.fori_loop(0, SEQ // 2, loop_body, 0)

    out_wait(SEQ - 2, out0, osem0)
    out_wait(SEQ - 1, out1, osem1)


def kernel(token_tensor, table, PE):
    tok4d = (token_tensor.astype(jnp.int32).T
             .reshape(SHI, 8, NW, BLK).transpose(0, 2, 1, 3))
    out5d, pad_idxs = _emb_kernel(table, tok4d, PE)
    out = out5d.transpose(2, 4, 0, 1, 3).reshape(BATCH, SEQ, DIM)
    return out, pad_idxs


# speedup vs baseline: 2.4640x; 1.0208x over previous
import functools

import jax
import jax.numpy as jnp
from jax import lax
from jax.experimental import pallas as pl
from jax.experimental.pallas import tpu as pltpu
from jax.experimental.pallas import tpu_sc as plsc

VOCAB = 1000000
DIM = 64
SEQ = 200
BATCH = 4096

NUM_CORES = 2
NUM_SUBCORES = 16
NW = NUM_CORES * NUM_SUBCORES
BLK = BATCH // NW
SHI = SEQ // 8

_mesh = plsc.VectorSubcoreMesh(
    core_axis_name="c", subcore_axis_name="s",
    num_cores=NUM_CORES, num_subcores=NUM_SUBCORES)


@functools.partial(
    pl.kernel,
    out_type=[
        jax.ShapeDtypeStruct((SEQ, DIM // 8, NW, 8, BLK), jnp.float32),
        jax.ShapeDtypeStruct((BATCH,), jnp.int32),
    ],
    mesh=_mesh,
    compiler_params=pltpu.CompilerParams(
        needs_layout_passes=False, use_tc_tiling_on_sc=False),
    scratch_types=[
        pltpu.VMEM((SEQ, BLK), jnp.int32),
        pltpu.VMEM((SEQ, DIM), jnp.float32),
        pltpu.VMEM((BLK, 2 * DIM), jnp.float32),
        pltpu.VMEM((BLK, 2 * DIM), jnp.float32),
        pltpu.VMEM((BLK, DIM + 1), jnp.float32),
        pltpu.VMEM((DIM, BLK), jnp.float32),
        pltpu.VMEM((DIM, BLK), jnp.float32),
        pltpu.VMEM((BLK,), jnp.int32),
        pltpu.SemaphoreType.DMA,
        pltpu.SemaphoreType.DMA,
        pltpu.SemaphoreType.DMA,
        pltpu.SemaphoreType.DMA,
        pltpu.SemaphoreType.DMA,
    ],
)
def _emb_kernel(table_hbm, tok_hbm, pe_hbm, out_hbm, pad_hbm,
                idx_v, pe_v, in0, in1, inp, out0, out1, pad_v,
                stsem, gsem0, gsem1, osem0, osem1):
    cid = lax.axis_index("c")
    sid = lax.axis_index("s")
    wid = sid * NUM_CORES + cid

    for shi in range(SHI):
        pltpu.make_async_copy(
            tok_hbm.at[shi, wid], idx_v.at[pl.ds(shi * 8, 8)], stsem).start()
    for shi in range(SHI):
        pltpu.make_async_copy(
            tok_hbm.at[shi, wid], idx_v.at[pl.ds(shi * 8, 8)], stsem).wait()

    def gather_start(s, buf, sem):
        pltpu.make_async_copy(table_hbm.at[idx_v.at[s]], buf, sem).start()

    def gather_wait(s, buf, sem):
        pltpu.make_async_copy(table_hbm.at[idx_v.at[s]], buf, sem).wait()

    def out_start(s, buf, sem):
        for dhi in range(DIM // 8):
            pltpu.make_async_copy(
                buf.at[pl.ds(dhi * 8, 8)], out_hbm.at[s, dhi, wid], sem).start()

    def out_wait(s, buf, sem):
        for dhi in range(DIM // 8):
            pltpu.make_async_copy(
                buf.at[pl.ds(dhi * 8, 8)], out_hbm.at[s, dhi, wid], sem).wait()

    gather_start(0, in0, gsem0)
    gather_start(1, in1, gsem1)

    pltpu.sync_copy(pe_hbm, pe_v)

    def pad_body(s, best):
        new = []
        for g in range(BLK // 16):
            tok = idx_v[s, pl.ds(16 * g, 16)]
            new.append(jnp.maximum(best[g], jnp.where(tok != 0, s, 0)))
        return tuple(new)

    best = lax.fori_loop(
        0, SEQ, pad_body,
        tuple(jnp.zeros((16,), jnp.int32) for _ in range(BLK // 16)))
    for g in range(BLK // 16):
        pad_v[pl.ds(16 * g, 16)] = best[g] + 1
    pltpu.sync_copy(pad_v, pad_hbm.at[pl.ds(wid * BLK, BLK)])

    iota16 = lax.iota(jnp.int32, 16)

    def compute(s, inb, outb):
        pe_c = [pe_v[s, pl.ds(16 * c, 16)] for c in range(DIM // 16)]

        @plsc.parallel_loop(0, BLK, 1, unroll=8)
        def r_body(r):
            for c in range(DIM // 16):
                sl = pl.ds(16 * c, 16)
                inp[r, sl] = inb[r, sl] * 8.0 + pe_c[c]

        @plsc.parallel_loop(0, DIM, 1, unroll=8)
        def d_body(d):
            col = jnp.full((16,), d, jnp.int32)
            for g in range(BLK // 16):
                row = 16 * g + iota16
                outb[d, pl.ds(16 * g, 16)] = plsc.load_gather(inp, [row, col])

    def s_phase(s, inb, outb, gsem, osem):
        gather_wait(s, inb, gsem)

        @pl.when(s >= 2)
        def _():
            out_wait(s, outb, osem)

        compute(s, inb, outb)

        @pl.when(s + 2 < SEQ)
        def _():
            gather_start(s + 2, inb, gsem)

        out_start(s, outb, osem)

    def loop_body(ss, _):
        s_phase(2 * ss, in0, out0, gsem0, osem0)
        s_phase(2 * ss + 1, in1, out1, gsem1, osem1)
        return 0

    lax.fori_loop(0, SEQ // 2, loop_body, 0)

    out_wait(SEQ - 2, out0, osem0)
    out_wait(SEQ - 1, out1, osem1)


def kernel(token_tensor, table, PE):
    tok4d = (token_tensor.astype(jnp.int32).T
             .reshape(SHI, 8, NW, BLK).transpose(0, 2, 1, 3))
    table_pad = jnp.pad(table, ((0, 0), (0, DIM)))
    out5d, pad_idxs = _emb_kernel(table_pad, tok4d, PE)
    out = out5d.transpose(2, 4, 0, 1, 3).reshape(BATCH, SEQ, DIM)
    return out, pad_idxs


# speedup vs baseline: 3.4020x; 1.3807x over previous
import functools

import jax
import jax.numpy as jnp
from jax import lax
from jax.experimental import pallas as pl
from jax.experimental.pallas import tpu as pltpu
from jax.experimental.pallas import tpu_sc as plsc

VOCAB = 1000000
DIM = 64
SEQ = 200
BATCH = 4096

NUM_CORES = 2
NUM_SUBCORES = 16
NW = NUM_CORES * NUM_SUBCORES
BLK = BATCH // NW
SHI = SEQ // 8

_mesh = plsc.VectorSubcoreMesh(
    core_axis_name="c", subcore_axis_name="s",
    num_cores=NUM_CORES, num_subcores=NUM_SUBCORES)

NB = VOCAB // 128
K_ITERS = (NB + NW - 1) // NW


@functools.partial(
    pl.kernel,
    out_type=jax.ShapeDtypeStruct((VOCAB, 2 * DIM), jnp.float32),
    mesh=_mesh,
    compiler_params=pltpu.CompilerParams(
        needs_layout_passes=False, use_tc_tiling_on_sc=True),
    scratch_types=[
        pltpu.VMEM((DIM, 128), jnp.float32),
        pltpu.VMEM((DIM, 128), jnp.float32),
        pltpu.VMEM((DIM * 129,), jnp.float32),
        pltpu.VMEM((128, 128), jnp.float32),
        pltpu.VMEM((128, 128), jnp.float32),
        pltpu.SemaphoreType.DMA,
        pltpu.SemaphoreType.DMA,
        pltpu.SemaphoreType.DMA,
        pltpu.SemaphoreType.DMA,
    ],
)
def _relayout_kernel(tbl_t_hbm, tail_hbm, out_hbm,
                     tiles0, tiles1, inp, rows0, rows1,
                     isem0, isem1, osem0, osem1):
    cid = lax.axis_index("c")
    sid = lax.axis_index("s")
    wid = sid * NUM_CORES + cid

    def block_of(k):
        return jnp.minimum(wid + NW * k, NB - 1)

    def in_start(k, tiles, sem):
        bi = block_of(k)
        for dhi in range(DIM // 8):
            pltpu.make_async_copy(
                tbl_t_hbm.at[pl.ds(8 * dhi, 8), pl.ds(128 * bi, 128)],
                tiles.at[pl.ds(8 * dhi, 8)], sem).start()

    def in_wait(k, tiles, sem):
        bi = block_of(k)
        for dhi in range(DIM // 8):
            pltpu.make_async_copy(
                tbl_t_hbm.at[pl.ds(8 * dhi, 8), pl.ds(128 * bi, 128)],
                tiles.at[pl.ds(8 * dhi, 8)], sem).wait()

    def out_start(k, rows, sem):
        bi = block_of(k)
        pltpu.make_async_copy(
            rows, out_hbm.at[pl.ds(128 * bi, 128)], sem).start()

    def out_wait(k, rows, sem):
        bi = block_of(k)
        pltpu.make_async_copy(
            rows, out_hbm.at[pl.ds(128 * bi, 128)], sem).wait()

    in_start(0, tiles0, isem0)
    in_start(1, tiles1, isem1)

    @pl.when(wid == 0)
    def _():
        pltpu.sync_copy(tail_hbm, out_hbm.at[pl.ds(NB * 128, 128 - DIM)])

    iota16 = lax.iota(jnp.int32, 16)

    def compute(tiles, rows):
        @plsc.parallel_loop(0, DIM, 1, unroll=8)
        def r_body(d):
            for c in range(8):
                inp[pl.ds(129 * d + 16 * c, 16)] = tiles[d, pl.ds(16 * c, 16)]

        @plsc.parallel_loop(0, 128, 1, unroll=4)
        def t_body(t):
            for c in range(DIM // 16):
                idx = (129 * 16 * c + t) + 129 * iota16
                rows[t, pl.ds(16 * c, 16)] = plsc.load_gather(inp, [idx])

    def phase(k, tiles, rows, isem, osem):
        in_wait(k, tiles, isem)

        @pl.when(k >= 2)
        def _():
            out_wait(k, rows, osem)

        compute(tiles, rows)

        @pl.when(k + 2 < 2 * ((K_ITERS + 1) // 2))
        def _():
            in_start(k + 2, tiles, isem)

        out_start(k, rows, osem)

    def loop_body(kk, _):
        phase(2 * kk, tiles0, rows0, isem0, osem0)
        phase(2 * kk + 1, tiles1, rows1, isem1, osem1)
        return 0

    half = (K_ITERS + 1) // 2
    lax.fori_loop(0, half, loop_body, 0)

    out_wait(2 * half - 2, rows0, osem0)
    out_wait(2 * half - 1, rows1, osem1)


@functools.partial(
    pl.kernel,
    out_type=[
        jax.ShapeDtypeStruct((SEQ, DIM // 8, NW, 8, BLK), jnp.float32),
        jax.ShapeDtypeStruct((BATCH,), jnp.int32),
    ],
    mesh=_mesh,
    compiler_params=pltpu.CompilerParams(
        needs_layout_passes=False, use_tc_tiling_on_sc=False),
    scratch_types=[
        pltpu.VMEM((SEQ, BLK), jnp.int32),
        pltpu.VMEM((SEQ, DIM), jnp.float32),
        pltpu.VMEM((BLK, 2 * DIM), jnp.float32),
        pltpu.VMEM((BLK, 2 * DIM), jnp.float32),
        pltpu.VMEM((BLK, DIM + 1), jnp.float32),
        pltpu.VMEM((DIM, BLK), jnp.float32),
        pltpu.VMEM((DIM, BLK), jnp.float32),
        pltpu.VMEM((BLK,), jnp.int32),
        pltpu.SemaphoreType.DMA,
        pltpu.SemaphoreType.DMA,
        pltpu.SemaphoreType.DMA,
        pltpu.SemaphoreType.DMA,
        pltpu.SemaphoreType.DMA,
    ],
)
def _emb_kernel(table_hbm, tok_hbm, pe_hbm, out_hbm, pad_hbm,
                idx_v, pe_v, in0, in1, inp, out0, out1, pad_v,
                stsem, gsem0, gsem1, osem0, osem1):
    cid = lax.axis_index("c")
    sid = lax.axis_index("s")
    wid = sid * NUM_CORES + cid

    for shi in range(SHI):
        pltpu.make_async_copy(
            tok_hbm.at[shi, wid], idx_v.at[pl.ds(shi * 8, 8)], stsem).start()
    for shi in range(SHI):
        pltpu.make_async_copy(
            tok_hbm.at[shi, wid], idx_v.at[pl.ds(shi * 8, 8)], stsem).wait()

    def gather_start(s, buf, sem):
        pltpu.make_async_copy(table_hbm.at[idx_v.at[s]], buf, sem).start()

    def gather_wait(s, buf, sem):
        pltpu.make_async_copy(table_hbm.at[idx_v.at[s]], buf, sem).wait()

    def out_start(s, buf, sem):
        for dhi in range(DIM // 8):
            pltpu.make_async_copy(
                buf.at[pl.ds(dhi * 8, 8)], out_hbm.at[s, dhi, wid], sem).start()

    def out_wait(s, buf, sem):
        for dhi in range(DIM // 8):
            pltpu.make_async_copy(
                buf.at[pl.ds(dhi * 8, 8)], out_hbm.at[s, dhi, wid], sem).wait()

    gather_start(0, in0, gsem0)
    gather_start(1, in1, gsem1)

    pltpu.sync_copy(pe_hbm, pe_v)

    def pad_body(s, best):
        new = []
        for g in range(BLK // 16):
            tok = idx_v[s, pl.ds(16 * g, 16)]
            new.append(jnp.maximum(best[g], jnp.where(tok != 0, s, 0)))
        return tuple(new)

    best = lax.fori_loop(
        0, SEQ, pad_body,
        tuple(jnp.zeros((16,), jnp.int32) for _ in range(BLK // 16)))
    for g in range(BLK // 16):
        pad_v[pl.ds(16 * g, 16)] = best[g] + 1
    pltpu.sync_copy(pad_v, pad_hbm.at[pl.ds(wid * BLK, BLK)])

    iota16 = lax.iota(jnp.int32, 16)

    def compute(s, inb, outb):
        pe_c = [pe_v[s, pl.ds(16 * c, 16)] for c in range(DIM // 16)]

        @plsc.parallel_loop(0, BLK, 1, unroll=8)
        def r_body(r):
            for c in range(DIM // 16):
                sl = pl.ds(16 * c, 16)
                inp[r, sl] = inb[r, sl] * 8.0 + pe_c[c]

        @plsc.parallel_loop(0, DIM, 1, unroll=8)
        def d_body(d):
            col = jnp.full((16,), d, jnp.int32)
            for g in range(BLK // 16):
                row = 16 * g + iota16
                outb[d, pl.ds(16 * g, 16)] = plsc.load_gather(inp, [row, col])

    def s_phase(s, inb, outb, gsem, osem):
        gather_wait(s, inb, gsem)

        @pl.when(s >= 2)
        def _():
            out_wait(s, outb, osem)

        compute(s, inb, outb)

        @pl.when(s + 2 < SEQ)
        def _():
            gather_start(s + 2, inb, gsem)

        out_start(s, outb, osem)

    def loop_body(ss, _):
        s_phase(2 * ss, in0, out0, gsem0, osem0)
        s_phase(2 * ss + 1, in1, out1, gsem1, osem1)
        return 0

    lax.fori_loop(0, SEQ // 2, loop_body, 0)

    out_wait(SEQ - 2, out0, osem0)
    out_wait(SEQ - 1, out1, osem1)


def kernel(token_tensor, table, PE):
    tok4d = (token_tensor.astype(jnp.int32).T
             .reshape(SHI, 8, NW, BLK).transpose(0, 2, 1, 3))
    tail = jnp.pad(table[NB * 128:], ((0, 0), (0, DIM)))
    table_pad = _relayout_kernel(table.T, tail)
    out5d, pad_idxs = _emb_kernel(table_pad, tok4d, PE)
    out = out5d.transpose(2, 4, 0, 1, 3).reshape(BATCH, SEQ, DIM)
    return out, pad_idxs


# speedup vs baseline: 3.7976x; 1.1163x over previous
import functools

import jax
import jax.numpy as jnp
from jax import lax
from jax.experimental import pallas as pl
from jax.experimental.pallas import tpu as pltpu
from jax.experimental.pallas import tpu_sc as plsc

VOCAB = 1000000
DIM = 64
SEQ = 200
BATCH = 4096

NUM_CORES = 2
NUM_SUBCORES = 16
NW = NUM_CORES * NUM_SUBCORES
BLK = BATCH // NW
SHI = SEQ // 8

_mesh = plsc.VectorSubcoreMesh(
    core_axis_name="c", subcore_axis_name="s",
    num_cores=NUM_CORES, num_subcores=NUM_SUBCORES)

NB = VOCAB // 128
K_ITERS = (NB + NW - 1) // NW


@functools.partial(
    pl.kernel,
    out_type=jax.ShapeDtypeStruct((VOCAB, 2 * DIM), jnp.float32),
    mesh=_mesh,
    compiler_params=pltpu.CompilerParams(
        needs_layout_passes=False, use_tc_tiling_on_sc=True),
    scratch_types=[
        pltpu.VMEM((DIM, 128), jnp.float32),
        pltpu.VMEM((DIM, 128), jnp.float32),
        pltpu.VMEM((DIM * 129,), jnp.float32),
        pltpu.VMEM((128, 128), jnp.float32),
        pltpu.VMEM((128, 128), jnp.float32),
        pltpu.SemaphoreType.DMA,
        pltpu.SemaphoreType.DMA,
        pltpu.SemaphoreType.DMA,
        pltpu.SemaphoreType.DMA,
    ],
)
def _relayout_kernel(tbl_t_hbm, tail_hbm, out_hbm,
                     tiles0, tiles1, inp, rows0, rows1,
                     isem0, isem1, osem0, osem1):
    cid = lax.axis_index("c")
    sid = lax.axis_index("s")
    wid = sid * NUM_CORES + cid

    def block_of(k):
        return jnp.minimum(wid + NW * k, NB - 1)

    def in_start(k, tiles, sem):
        bi = block_of(k)
        for dhi in range(DIM // 8):
            pltpu.make_async_copy(
                tbl_t_hbm.at[pl.ds(8 * dhi, 8), pl.ds(128 * bi, 128)],
                tiles.at[pl.ds(8 * dhi, 8)], sem).start()

    def in_wait(k, tiles, sem):
        bi = block_of(k)
        for dhi in range(DIM // 8):
            pltpu.make_async_copy(
                tbl_t_hbm.at[pl.ds(8 * dhi, 8), pl.ds(128 * bi, 128)],
                tiles.at[pl.ds(8 * dhi, 8)], sem).wait()

    def out_start(k, rows, sem):
        bi = block_of(k)
        pltpu.make_async_copy(
            rows, out_hbm.at[pl.ds(128 * bi, 128)], sem).start()

    def out_wait(k, rows, sem):
        bi = block_of(k)
        pltpu.make_async_copy(
            rows, out_hbm.at[pl.ds(128 * bi, 128)], sem).wait()

    in_start(0, tiles0, isem0)
    in_start(1, tiles1, isem1)

    @pl.when(wid == 0)
    def _():
        pltpu.sync_copy(tail_hbm, out_hbm.at[pl.ds(NB * 128, 128 - DIM)])

    iota16 = lax.iota(jnp.int32, 16)

    def compute(tiles, rows):
        @plsc.parallel_loop(0, DIM, 1, unroll=8)
        def r_body(d):
            for c in range(8):
                inp[pl.ds(129 * d + 16 * c, 16)] = tiles[d, pl.ds(16 * c, 16)]

        @plsc.parallel_loop(0, 128, 1, unroll=4)
        def t_body(t):
            for c in range(DIM // 16):
                idx = (129 * 16 * c + t) + 129 * iota16
                rows[t, pl.ds(16 * c, 16)] = plsc.load_gather(inp, [idx])

    def phase(k, tiles, rows, isem, osem):
        in_wait(k, tiles, isem)

        @pl.when(k >= 2)
        def _():
            out_wait(k, rows, osem)

        compute(tiles, rows)

        @pl.when(k + 2 < 2 * ((K_ITERS + 1) // 2))
        def _():
            in_start(k + 2, tiles, isem)

        out_start(k, rows, osem)

    def loop_body(kk, _):
        phase(2 * kk, tiles0, rows0, isem0, osem0)
        phase(2 * kk + 1, tiles1, rows1, isem1, osem1)
        return 0

    half = (K_ITERS + 1) // 2
    lax.fori_loop(0, half, loop_body, 0)

    out_wait(2 * half - 2, rows0, osem0)
    out_wait(2 * half - 1, rows1, osem1)


@functools.partial(
    pl.kernel,
    out_type=[
        jax.ShapeDtypeStruct((SEQ, DIM // 8, NW, 8, BLK), jnp.float32),
        jax.ShapeDtypeStruct((BATCH,), jnp.int32),
    ],
    mesh=_mesh,
    compiler_params=pltpu.CompilerParams(
        needs_layout_passes=False, use_tc_tiling_on_sc=False),
    scratch_types=[
        pltpu.VMEM((SEQ, BLK), jnp.int32),
        pltpu.VMEM((SEQ, DIM), jnp.float32),
        pltpu.VMEM((BLK, 2 * DIM), jnp.float32),
        pltpu.VMEM((BLK, 2 * DIM), jnp.float32),
        pltpu.VMEM((BLK, 2 * DIM), jnp.float32),
        pltpu.VMEM((BLK, DIM + 1), jnp.float32),
        pltpu.VMEM((DIM, BLK), jnp.float32),
        pltpu.VMEM((DIM, BLK), jnp.float32),
        pltpu.VMEM((DIM, BLK), jnp.float32),
        pltpu.VMEM((BLK,), jnp.int32),
        pltpu.SemaphoreType.DMA,
        pltpu.SemaphoreType.DMA,
        pltpu.SemaphoreType.DMA,
        pltpu.SemaphoreType.DMA,
        pltpu.SemaphoreType.DMA,
        pltpu.SemaphoreType.DMA,
        pltpu.SemaphoreType.DMA,
    ],
)
def _emb_kernel(table_hbm, tok_hbm, pe_hbm, out_hbm, pad_hbm,
                idx_v, pe_v, in0, in1, in2, inp, out0, out1, out2, pad_v,
                stsem, gsem0, gsem1, gsem2, osem0, osem1, osem2):
    cid = lax.axis_index("c")
    sid = lax.axis_index("s")
    wid = sid * NUM_CORES + cid

    for shi in range(SHI):
        pltpu.make_async_copy(
            tok_hbm.at[shi, wid], idx_v.at[pl.ds(shi * 8, 8)], stsem).start()
    for shi in range(SHI):
        pltpu.make_async_copy(
            tok_hbm.at[shi, wid], idx_v.at[pl.ds(shi * 8, 8)], stsem).wait()

    def gather_start(s, buf, sem):
        pltpu.make_async_copy(table_hbm.at[idx_v.at[s]], buf, sem).start()

    def gather_wait(s, buf, sem):
        pltpu.make_async_copy(table_hbm.at[idx_v.at[s]], buf, sem).wait()

    def out_start(s, buf, sem):
        for dhi in range(DIM // 8):
            pltpu.make_async_copy(
                buf.at[pl.ds(dhi * 8, 8)], out_hbm.at[s, dhi, wid], sem).start()

    def out_wait(s, buf, sem):
        for dhi in range(DIM // 8):
            pltpu.make_async_copy(
                buf.at[pl.ds(dhi * 8, 8)], out_hbm.at[s, dhi, wid], sem).wait()

    gather_start(0, in0, gsem0)
    gather_start(1, in1, gsem1)
    gather_start(2, in2, gsem2)

    pltpu.sync_copy(pe_hbm, pe_v)

    def pad_body(s, best):
        new = []
        for g in range(BLK // 16):
            tok = idx_v[s, pl.ds(16 * g, 16)]
            new.append(jnp.maximum(best[g], jnp.where(tok != 0, s, 0)))
        return tuple(new)

    best = lax.fori_loop(
        0, SEQ, pad_body,
        tuple(jnp.zeros((16,), jnp.int32) for _ in range(BLK // 16)))
    for g in range(BLK // 16):
        pad_v[pl.ds(16 * g, 16)] = best[g] + 1
    pltpu.sync_copy(pad_v, pad_hbm.at[pl.ds(wid * BLK, BLK)])

    iota16 = lax.iota(jnp.int32, 16)

    def compute(s, inb, outb):
        pe_c = [pe_v[s, pl.ds(16 * c, 16)] for c in range(DIM // 16)]

        @plsc.parallel_loop(0, BLK, 1, unroll=8)
        def r_body(r):
            for c in range(DIM // 16):
                sl = pl.ds(16 * c, 16)
                inp[r, sl] = inb[r, sl] * 8.0 + pe_c[c]

        @plsc.parallel_loop(0, DIM, 1, unroll=8)
        def d_body(d):
            col = jnp.full((16,), d, jnp.int32)
            for g in range(BLK // 16):
                row = 16 * g + iota16
                outb[d, pl.ds(16 * g, 16)] = plsc.load_gather(inp, [row, col])

    def s_phase(s, inb, outb, gsem, osem):
        gather_wait(s, inb, gsem)

        @pl.when(s >= 3)
        def _():
            out_wait(s, outb, osem)

        compute(s, inb, outb)

        @pl.when(s + 3 < SEQ)
        def _():
            gather_start(s + 3, inb, gsem)

        out_start(s, outb, osem)

    def loop_body(ss, _):
        s_phase(3 * ss, in0, out0, gsem0, osem0)
        s_phase(3 * ss + 1, in1, out1, gsem1, osem1)

        @pl.when(ss < SEQ // 3)
        def _():
            s_phase(3 * ss + 2, in2, out2, gsem2, osem2)

        return 0

    lax.fori_loop(0, SEQ // 3 + 1, loop_body, 0)

    out_wait(SEQ - 3, out2, osem2)
    out_wait(SEQ - 2, out0, osem0)
    out_wait(SEQ - 1, out1, osem1)


def kernel(token_tensor, table, PE):
    tok4d = (token_tensor.astype(jnp.int32).T
             .reshape(SHI, 8, NW, BLK).transpose(0, 2, 1, 3))
    tail = jnp.pad(table[NB * 128:], ((0, 0), (0, DIM)))
    table_pad = _relayout_kernel(table.T, tail)
    out5d, pad_idxs = _emb_kernel(table_pad, tok4d, PE)
    out = out5d.transpose(2, 4, 0, 1, 3).reshape(BATCH, SEQ, DIM)
    return out, pad_idxs


# speedup vs baseline: 3.8201x; 1.0059x over previous
import functools

import jax
import jax.numpy as jnp
from jax import lax
from jax.experimental import pallas as pl
from jax.experimental.pallas import tpu as pltpu
from jax.experimental.pallas import tpu_sc as plsc

VOCAB = 1000000
DIM = 64
SEQ = 200
BATCH = 4096

NUM_CORES = 2
NUM_SUBCORES = 16
NW = NUM_CORES * NUM_SUBCORES
BLK = BATCH // NW
SHI = SEQ // 8

_mesh = plsc.VectorSubcoreMesh(
    core_axis_name="c", subcore_axis_name="s",
    num_cores=NUM_CORES, num_subcores=NUM_SUBCORES)

NB = VOCAB // 128
K_ITERS = (NB + NW - 1) // NW


@functools.partial(
    pl.kernel,
    out_type=jax.ShapeDtypeStruct((VOCAB, 2 * DIM), jnp.float32),
    mesh=_mesh,
    compiler_params=pltpu.CompilerParams(
        needs_layout_passes=False, use_tc_tiling_on_sc=True),
    scratch_types=[
        pltpu.VMEM((DIM, 128), jnp.float32),
        pltpu.VMEM((DIM, 128), jnp.float32),
        pltpu.VMEM((DIM, 128), jnp.float32),
        pltpu.VMEM((DIM * 129,), jnp.float32),
        pltpu.VMEM((128, 128), jnp.float32),
        pltpu.VMEM((128, 128), jnp.float32),
        pltpu.VMEM((128, 128), jnp.float32),
        pltpu.SemaphoreType.DMA,
        pltpu.SemaphoreType.DMA,
        pltpu.SemaphoreType.DMA,
        pltpu.SemaphoreType.DMA,
        pltpu.SemaphoreType.DMA,
        pltpu.SemaphoreType.DMA,
    ],
)
def _relayout_kernel(tbl_t_hbm, tail_hbm, out_hbm,
                     tiles0, tiles1, tiles2, inp, rows0, rows1, rows2,
                     isem0, isem1, isem2, osem0, osem1, osem2):
    cid = lax.axis_index("c")
    sid = lax.axis_index("s")
    wid = sid * NUM_CORES + cid

    def block_of(k):
        return jnp.minimum(wid + NW * k, NB - 1)

    def in_start(k, tiles, sem):
        bi = block_of(k)
        for dhi in range(DIM // 8):
            pltpu.make_async_copy(
                tbl_t_hbm.at[pl.ds(8 * dhi, 8), pl.ds(128 * bi, 128)],
                tiles.at[pl.ds(8 * dhi, 8)], sem).start()

    def in_wait(k, tiles, sem):
        bi = block_of(k)
        for dhi in range(DIM // 8):
            pltpu.make_async_copy(
                tbl_t_hbm.at[pl.ds(8 * dhi, 8), pl.ds(128 * bi, 128)],
                tiles.at[pl.ds(8 * dhi, 8)], sem).wait()

    def out_start(k, rows, sem):
        bi = block_of(k)
        pltpu.make_async_copy(
            rows, out_hbm.at[pl.ds(128 * bi, 128)], sem).start()

    def out_wait(k, rows, sem):
        bi = block_of(k)
        pltpu.make_async_copy(
            rows, out_hbm.at[pl.ds(128 * bi, 128)], sem).wait()

    in_start(0, tiles0, isem0)
    in_start(1, tiles1, isem1)
    in_start(2, tiles2, isem2)

    @pl.when(wid == 0)
    def _():
        pltpu.sync_copy(tail_hbm, out_hbm.at[pl.ds(NB * 128, 128 - DIM)])

    iota16 = lax.iota(jnp.int32, 16)

    def compute(tiles, rows):
        @plsc.parallel_loop(0, DIM, 1, unroll=8)
        def r_body(d):
            for c in range(8):
                inp[pl.ds(129 * d + 16 * c, 16)] = tiles[d, pl.ds(16 * c, 16)]

        @plsc.parallel_loop(0, 128, 1, unroll=4)
        def t_body(t):
            for c in range(DIM // 16):
                idx = (129 * 16 * c + t) + 129 * iota16
                rows[t, pl.ds(16 * c, 16)] = plsc.load_gather(inp, [idx])

    n_phases = 3 * ((K_ITERS + 2) // 3)

    def phase(k, tiles, rows, isem, osem):
        in_wait(k, tiles, isem)

        @pl.when(k >= 3)
        def _():
            out_wait(k, rows, osem)

        compute(tiles, rows)

        @pl.when(k + 3 < n_phases)
        def _():
            in_start(k + 3, tiles, isem)

        out_start(k, rows, osem)

    def loop_body(kk, _):
        phase(3 * kk, tiles0, rows0, isem0, osem0)
        phase(3 * kk + 1, tiles1, rows1, isem1, osem1)
        phase(3 * kk + 2, tiles2, rows2, isem2, osem2)
        return 0

    lax.fori_loop(0, n_phases // 3, loop_body, 0)

    out_wait(n_phases - 3, rows0, osem0)
    out_wait(n_phases - 2, rows1, osem1)
    out_wait(n_phases - 1, rows2, osem2)


@functools.partial(
    pl.kernel,
    out_type=[
        jax.ShapeDtypeStruct((SEQ, DIM // 8, NW, 8, BLK), jnp.float32),
        jax.ShapeDtypeStruct((BATCH,), jnp.int32),
    ],
    mesh=_mesh,
    compiler_params=pltpu.CompilerParams(
        needs_layout_passes=False, use_tc_tiling_on_sc=False),
    scratch_types=[
        pltpu.VMEM((SEQ, BLK), jnp.int32),
        pltpu.VMEM((SEQ, DIM), jnp.float32),
        pltpu.VMEM((BLK, 2 * DIM), jnp.float32),
        pltpu.VMEM((BLK, 2 * DIM), jnp.float32),
        pltpu.VMEM((BLK, 2 * DIM), jnp.float32),
        pltpu.VMEM((BLK, DIM + 1), jnp.float32),
        pltpu.VMEM((DIM, BLK), jnp.float32),
        pltpu.VMEM((DIM, BLK), jnp.float32),
        pltpu.VMEM((DIM, BLK), jnp.float32),
        pltpu.VMEM((BLK,), jnp.int32),
        pltpu.SemaphoreType.DMA,
        pltpu.SemaphoreType.DMA,
        pltpu.SemaphoreType.DMA,
        pltpu.SemaphoreType.DMA,
        pltpu.SemaphoreType.DMA,
        pltpu.SemaphoreType.DMA,
        pltpu.SemaphoreType.DMA,
    ],
)
def _emb_kernel(table_hbm, tok_hbm, pe_hbm, out_hbm, pad_hbm,
                idx_v, pe_v, in0, in1, in2, inp, out0, out1, out2, pad_v,
                stsem, gsem0, gsem1, gsem2, osem0, osem1, osem2):
    cid = lax.axis_index("c")
    sid = lax.axis_index("s")
    wid = sid * NUM_CORES + cid

    for shi in range(SHI):
        pltpu.make_async_copy(
            tok_hbm.at[shi, wid], idx_v.at[pl.ds(shi * 8, 8)], stsem).start()
    for shi in range(SHI):
        pltpu.make_async_copy(
            tok_hbm.at[shi, wid], idx_v.at[pl.ds(shi * 8, 8)], stsem).wait()

    def gather_start(s, buf, sem):
        pltpu.make_async_copy(table_hbm.at[idx_v.at[s]], buf, sem).start()

    def gather_wait(s, buf, sem):
        pltpu.make_async_copy(table_hbm.at[idx_v.at[s]], buf, sem).wait()

    def out_start(s, buf, sem):
        for dhi in range(DIM // 8):
            pltpu.make_async_copy(
                buf.at[pl.ds(dhi * 8, 8)], out_hbm.at[s, dhi, wid], sem).start()

    def out_wait(s, buf, sem):
        for dhi in range(DIM // 8):
            pltpu.make_async_copy(
                buf.at[pl.ds(dhi * 8, 8)], out_hbm.at[s, dhi, wid], sem).wait()

    gather_start(0, in0, gsem0)
    gather_start(1, in1, gsem1)
    gather_start(2, in2, gsem2)

    pltpu.sync_copy(pe_hbm, pe_v)

    def pad_body(s, best):
        new = []
        for g in range(BLK // 16):
            tok = idx_v[s, pl.ds(16 * g, 16)]
            new.append(jnp.maximum(best[g], jnp.where(tok != 0, s, 0)))
        return tuple(new)

    best = lax.fori_loop(
        0, SEQ, pad_body,
        tuple(jnp.zeros((16,), jnp.int32) for _ in range(BLK // 16)))
    for g in range(BLK // 16):
        pad_v[pl.ds(16 * g, 16)] = best[g] + 1
    pltpu.sync_copy(pad_v, pad_hbm.at[pl.ds(wid * BLK, BLK)])

    iota16 = lax.iota(jnp.int32, 16)

    def compute(s, inb, outb):
        pe_c = [pe_v[s, pl.ds(16 * c, 16)] for c in range(DIM // 16)]

        @plsc.parallel_loop(0, BLK, 1, unroll=8)
        def r_body(r):
            for c in range(DIM // 16):
                sl = pl.ds(16 * c, 16)
                inp[r, sl] = inb[r, sl] * 8.0 + pe_c[c]

        @plsc.parallel_loop(0, DIM, 1, unroll=8)
        def d_body(d):
            col = jnp.full((16,), d, jnp.int32)
            for g in range(BLK // 16):
                row = 16 * g + iota16
                outb[d, pl.ds(16 * g, 16)] = plsc.load_gather(inp, [row, col])

    def s_phase(s, inb, outb, gsem, osem):
        gather_wait(s, inb, gsem)

        @pl.when(s >= 3)
        def _():
            out_wait(s, outb, osem)

        compute(s, inb, outb)

        @pl.when(s + 3 < SEQ)
        def _():
            gather_start(s + 3, inb, gsem)

        out_start(s, outb, osem)

    def loop_body(ss, _):
        s_phase(3 * ss, in0, out0, gsem0, osem0)
        s_phase(3 * ss + 1, in1, out1, gsem1, osem1)

        @pl.when(ss < SEQ // 3)
        def _():
            s_phase(3 * ss + 2, in2, out2, gsem2, osem2)

        return 0

    lax.fori_loop(0, SEQ // 3 + 1, loop_body, 0)

    out_wait(SEQ - 3, out2, osem2)
    out_wait(SEQ - 2, out0, osem0)
    out_wait(SEQ - 1, out1, osem1)


def kernel(token_tensor, table, PE):
    tok4d = (token_tensor.astype(jnp.int32).T
             .reshape(SHI, 8, NW, BLK).transpose(0, 2, 1, 3))
    tail = jnp.pad(table[NB * 128:], ((0, 0), (0, DIM)))
    table_pad = _relayout_kernel(table.T, tail)
    out5d, pad_idxs = _emb_kernel(table_pad, tok4d, PE)
    out = out5d.transpose(2, 4, 0, 1, 3).reshape(BATCH, SEQ, DIM)
    return out, pad_idxs
